# Initial kernel scaffold; baseline (speedup 1.0000x reference)
import jax
import jax.numpy as jnp
from jax.experimental import pallas as pl


def kernel(x, edge_index, batch, W_gcn, b_gcn, W_gat, a_src, a_dst, b_gat, W_lin, b_lin):
    raise NotImplementedError("write your pallas kernel here")


# speedup vs baseline: 21.4371x; 21.4371x over previous
import functools

import jax
import jax.numpy as jnp
from jax import lax
from jax.experimental import pallas as pl
from jax.experimental.pallas import tpu as pltpu
from jax.experimental.pallas import tpu_sc as plsc

N = 10000
D = 128
G = 64

NC = 2
NS = 16
L = 16
NW = NC * NS

CHUNK = 128
N_PAD = 10240
RPS = N_PAD // NS

_mesh = plsc.VectorSubcoreMesh(core_axis_name="c", subcore_axis_name="s")
_SC_PARAMS = pltpu.CompilerParams(needs_layout_passes=False)


def _worker_base(chunks_per_w):
    c = lax.axis_index("c")
    s = lax.axis_index("s")
    wid = s * NC + c
    return wid * (chunks_per_w * CHUNK)


def _make_deg_kernel(chunks_per_w):
    @functools.partial(
        pl.kernel,
        out_type=jax.ShapeDtypeStruct((NC, N_PAD), jnp.float32),
        mesh=_mesh,
        compiler_params=_SC_PARAMS,
        scratch_types=[
            pltpu.VMEM_SHARED((N_PAD,), jnp.float32),
            pltpu.VMEM((CHUNK,), jnp.int32),
            pltpu.VMEM((CHUNK,), jnp.float32),
        ],
    )
    def deg_kernel(dst_hbm, z1_hbm, out_hbm, deg_sh, idx_v, ones_v):
        c = lax.axis_index("c")
        s = lax.axis_index("s")
        r0 = s * RPS
        pltpu.sync_copy(z1_hbm.at[pl.ds(r0, RPS)], deg_sh.at[pl.ds(r0, RPS)])
        for i in range(CHUNK // L):
            ones_v[pl.ds(i * L, L)] = jnp.ones((L,), jnp.float32)
        plsc.subcore_barrier()
        base = _worker_base(chunks_per_w)

        def body(k, carry):
            off = base + k * CHUNK
            pltpu.sync_copy(dst_hbm.at[pl.ds(off, CHUNK)], idx_v)
            pltpu.sync_copy(ones_v, deg_sh.at[idx_v], add=True)
            return carry

        lax.fori_loop(0, chunks_per_w, body, 0)
        plsc.subcore_barrier()
        pltpu.sync_copy(deg_sh.at[pl.ds(r0, RPS)],
                        out_hbm.at[c, pl.ds(r0, RPS)])

    return deg_kernel


def _make_gcn_kernel(chunks_per_w):
    @functools.partial(
        pl.kernel,
        out_type=jax.ShapeDtypeStruct((NC, N_PAD, D), jnp.float32),
        mesh=_mesh,
        compiler_params=_SC_PARAMS,
        scratch_types=[
            pltpu.VMEM_SHARED((N_PAD, D), jnp.float32),
            pltpu.VMEM((CHUNK,), jnp.int32),
            pltpu.VMEM((CHUNK,), jnp.int32),
            pltpu.VMEM((CHUNK, D), jnp.float32),
            pltpu.SemaphoreType.DMA,
        ],
    )
    def gcn_kernel(hs_hbm, src_hbm, dst_hbm, z2_hbm, out_hbm,
                   acc_sh, src_v, dst_v, rows_v, sem):
        c = lax.axis_index("c")
        s = lax.axis_index("s")
        r0 = s * RPS
        pltpu.sync_copy(z2_hbm.at[pl.ds(r0, RPS)], acc_sh.at[pl.ds(r0, RPS)])
        plsc.subcore_barrier()
        base = _worker_base(chunks_per_w)

        def body(k, carry):
            off = base + k * CHUNK
            pltpu.sync_copy(src_hbm.at[pl.ds(off, CHUNK)], src_v)
            pltpu.sync_copy(dst_hbm.at[pl.ds(off, CHUNK)], dst_v)
            pltpu.async_copy(hs_hbm.at[src_v], rows_v, sem).wait()
            pltpu.sync_copy(rows_v, acc_sh.at[dst_v], add=True)
            return carry

        lax.fori_loop(0, chunks_per_w, body, 0)
        plsc.subcore_barrier()
        pltpu.sync_copy(acc_sh.at[pl.ds(r0, RPS)],
                        out_hbm.at[c, pl.ds(r0, RPS)])

    return gcn_kernel


def _make_gat_kernel(chunks_per_w):
    @functools.partial(
        pl.kernel,
        out_type=[
            jax.ShapeDtypeStruct((NC, N_PAD, D), jnp.float32),
            jax.ShapeDtypeStruct((NC, N_PAD), jnp.float32),
        ],
        mesh=_mesh,
        compiler_params=_SC_PARAMS,
        scratch_types=[
            pltpu.VMEM_SHARED((N_PAD, D), jnp.float32),
            pltpu.VMEM_SHARED((N_PAD,), jnp.float32),
            pltpu.VMEM((N_PAD,), jnp.float32),
            pltpu.VMEM((N_PAD,), jnp.float32),
            pltpu.VMEM((CHUNK,), jnp.int32),
            pltpu.VMEM((CHUNK,), jnp.int32),
            pltpu.VMEM((CHUNK,), jnp.float32),
            pltpu.VMEM((CHUNK, D), jnp.float32),
            pltpu.SemaphoreType.DMA,
        ],
    )
    def gat_kernel(h2_hbm, ssrc_hbm, sdst_hbm, src_hbm, dst_hbm, z2_hbm,
                   z1_hbm, num_hbm, den_hbm,
                   num_sh, den_sh, ssrc_v, sdst_v, src_v, dst_v, w_v,
                   rows_v, sem):
        c = lax.axis_index("c")
        s = lax.axis_index("s")
        r0 = s * RPS
        pltpu.sync_copy(z2_hbm.at[pl.ds(r0, RPS)], num_sh.at[pl.ds(r0, RPS)])
        pltpu.sync_copy(z1_hbm.at[pl.ds(r0, RPS)], den_sh.at[pl.ds(r0, RPS)])
        pltpu.sync_copy(ssrc_hbm, ssrc_v)
        pltpu.sync_copy(sdst_hbm, sdst_v)
        plsc.subcore_barrier()
        base = _worker_base(chunks_per_w)

        def body(k, carry):
            off = base + k * CHUNK
            pltpu.sync_copy(src_hbm.at[pl.ds(off, CHUNK)], src_v)
            pltpu.sync_copy(dst_hbm.at[pl.ds(off, CHUNK)], dst_v)
            gather = pltpu.async_copy(h2_hbm.at[src_v], rows_v, sem)
            for i in range(CHUNK // L):
                si = src_v[pl.ds(i * L, L)]
                di = dst_v[pl.ds(i * L, L)]
                ss = plsc.load_gather(ssrc_v, [si])
                sd = plsc.load_gather(sdst_v, [di])
                e = ss + sd
                e = jnp.maximum(e, 0.2 * e)
                w_v[pl.ds(i * L, L)] = jnp.exp(e)
            pltpu.sync_copy(w_v, den_sh.at[dst_v], add=True)
            gather.wait()

            def scale(i, carry2):
                wi = plsc.load_gather(w_v, [jnp.full((L,), i, jnp.int32)])
                for j in range(D // L):
                    sl = pl.ds(j * L, L)
                    rows_v[i, sl] = rows_v[i, sl] * wi
                return carry2

            lax.fori_loop(0, CHUNK, scale, 0)
            pltpu.sync_copy(rows_v, num_sh.at[dst_v], add=True)
            return carry

        lax.fori_loop(0, chunks_per_w, body, 0)
        plsc.subcore_barrier()
        pltpu.sync_copy(num_sh.at[pl.ds(r0, RPS)],
                        num_hbm.at[c, pl.ds(r0, RPS)])
        pltpu.sync_copy(den_sh.at[pl.ds(r0, RPS)],
                        den_hbm.at[c, pl.ds(r0, RPS)])

    return gat_kernel


_BLK = 1000


def _tc1_body(x_ref, w_ref, degT_ref, hs_ref):
    deg = degT_ref[:, 0:1] + degT_ref[:, 1:2]
    dinv = jnp.where(deg > 0, lax.rsqrt(jnp.maximum(deg, 1e-12)), 0.0)
    h = jnp.dot(x_ref[...], w_ref[...], preferred_element_type=jnp.float32)
    hs_ref[...] = h * dinv


def _tc1(x, w_gcn, degT):
    return pl.pallas_call(
        _tc1_body,
        grid=(N // _BLK,),
        in_specs=[
            pl.BlockSpec((_BLK, D), lambda i: (i, 0)),
            pl.BlockSpec((D, D), lambda i: (0, 0)),
            pl.BlockSpec((_BLK, 2), lambda i: (i, 0)),
        ],
        out_specs=pl.BlockSpec((_BLK, D), lambda i: (i, 0)),
        out_shape=jax.ShapeDtypeStruct((N, D), jnp.float32),
    )(x, w_gcn, degT)


def _tc2_body(accp_ref, degT_ref, bg_ref, wgat_ref, a2_ref,
              h2_ref, ss_ref, sd_ref):
    deg = degT_ref[:, 0:1] + degT_ref[:, 1:2]
    dinv = jnp.where(deg > 0, lax.rsqrt(jnp.maximum(deg, 1e-12)), 0.0)
    y = (accp_ref[0] + accp_ref[1]) * dinv + bg_ref[...]
    h = jnp.maximum(y, 0.01 * y)
    h2 = jnp.dot(h, wgat_ref[...], preferred_element_type=jnp.float32)
    h2_ref[...] = h2
    s2 = jnp.dot(h2, a2_ref[...], preferred_element_type=jnp.float32)
    ss_ref[...] = s2[:, 0:1]
    sd_ref[...] = s2[:, 1:2]


def _tc2(accp, degT, b_gcn, w_gat, a2):
    blk = 1024
    return pl.pallas_call(
        _tc2_body,
        grid=(N_PAD // blk,),
        in_specs=[
            pl.BlockSpec((2, blk, D), lambda i: (0, i, 0)),
            pl.BlockSpec((blk, 2), lambda i: (i, 0)),
            pl.BlockSpec((1, D), lambda i: (0, 0)),
            pl.BlockSpec((D, D), lambda i: (0, 0)),
            pl.BlockSpec((D, 2), lambda i: (0, 0)),
        ],
        out_specs=[
            pl.BlockSpec((blk, D), lambda i: (i, 0)),
            pl.BlockSpec((blk, 1), lambda i: (i, 0)),
            pl.BlockSpec((blk, 1), lambda i: (i, 0)),
        ],
        out_shape=[
            jax.ShapeDtypeStruct((N_PAD, D), jnp.float32),
            jax.ShapeDtypeStruct((N_PAD, 1), jnp.float32),
            jax.ShapeDtypeStruct((N_PAD, 1), jnp.float32),
        ],
    )(accp, degT, b_gcn, w_gat, a2)


def _tc3_body(nump_ref, denT_ref, bg_ref, batch_ref, wlin_ref, blin_ref,
              out_ref, sums_ref, cnts_ref):
    i = pl.program_id(0)

    @pl.when(i == 0)
    def _():
        sums_ref[...] = jnp.zeros_like(sums_ref)
        cnts_ref[...] = jnp.zeros_like(cnts_ref)

    den = denT_ref[:, 0:1] + denT_ref[:, 1:2]
    y = (nump_ref[0] + nump_ref[1]) / jnp.maximum(den, 1e-16) + bg_ref[...]
    h3 = jnp.maximum(y, 0.01 * y)
    b = batch_ref[0]
    gids = lax.broadcasted_iota(jnp.int32, (G, _BLK), 0)
    onehot = (gids == b).astype(jnp.float32)
    sums_ref[...] += jnp.dot(onehot, h3, preferred_element_type=jnp.float32)
    cnts_ref[...] += jnp.sum(onehot, axis=1, keepdims=True)

    @pl.when(i == pl.num_programs(0) - 1)
    def _():
        pooled = sums_ref[...] / jnp.maximum(cnts_ref[...], 1.0)
        out_ref[...] = (
            jnp.dot(pooled, wlin_ref[...], preferred_element_type=jnp.float32)
            + blin_ref[...]
        )


def _tc3(nump, denT, b_gat, batch2d, w_lin, b_lin):
    return pl.pallas_call(
        _tc3_body,
        grid=(N // _BLK,),
        in_specs=[
            pl.BlockSpec((2, _BLK, D), lambda i: (0, i, 0)),
            pl.BlockSpec((_BLK, 2), lambda i: (i, 0)),
            pl.BlockSpec((1, D), lambda i: (0, 0)),
            pl.BlockSpec((1, 1, _BLK), lambda i: (i, 0, 0)),
            pl.BlockSpec((D, 1), lambda i: (0, 0)),
            pl.BlockSpec((1, 1), lambda i: (0, 0)),
        ],
        out_specs=pl.BlockSpec((G, 1), lambda i: (0, 0)),
        out_shape=jax.ShapeDtypeStruct((G, 1), jnp.float32),
        scratch_shapes=[
            pltpu.VMEM((G, D), jnp.float32),
            pltpu.VMEM((G, 1), jnp.float32),
        ],
    )(nump, denT, b_gat, batch2d, w_lin, b_lin)


@jax.jit
def kernel(x, edge_index, batch, W_gcn, b_gcn, W_gat, a_src, a_dst, b_gat,
           W_lin, b_lin):
    E = edge_index.shape[1]
    etot = E + N
    chunks_per_w = -(-etot // (NW * CHUNK))
    e_pad = chunks_per_w * CHUNK * NW
    ar = jnp.arange(N, dtype=jnp.int32)
    src = jnp.concatenate(
        [edge_index[0], ar,
         jnp.zeros((e_pad - etot,), jnp.int32)])
    dst = jnp.concatenate(
        [edge_index[1], ar,
         jnp.full((e_pad - etot,), N, jnp.int32)])

    z1 = jnp.zeros((N_PAD,), jnp.float32)
    z2 = jnp.zeros((N_PAD, D), jnp.float32)

    degp = _make_deg_kernel(chunks_per_w)(dst, z1)
    degT = degp.T

    hs = _tc1(x, W_gcn, degT)

    accp = _make_gcn_kernel(chunks_per_w)(hs, src, dst, z2)

    a2 = jnp.stack([a_src, a_dst], axis=1)
    h2, ss2, sd2 = _tc2(accp, degT, b_gcn.reshape(1, D), W_gat, a2)

    nump, denp = _make_gat_kernel(chunks_per_w)(
        h2, ss2.reshape(N_PAD), sd2.reshape(N_PAD), src, dst, z2, z1)

    out = _tc3(nump, denp.T, b_gat.reshape(1, D),
               batch.reshape(N // _BLK, 1, _BLK),
               W_lin, b_lin.reshape(1, 1))
    return out
